# baseline (speedup 1.0000x reference)
import functools
import jax
import jax.numpy as jnp
from jax import lax
from jax.experimental import pallas as pl
from jax.experimental.pallas import tpu as pltpu
from jax.experimental.pallas import tpu_sc as plsc

_N = 10000
_H = 64
_NC = 2
_NS = 16
_NW = _NC * _NS
_CHUNK = 128
_CPW = 80
_EPAD = _NW * _CPW * _CHUNK
_ACC = 10240
_SINK = _N
_ZPC = _ACC // _NS // _CHUNK
_DPS = _ACC // _NS
_CW = 16
_BN = 2000

_mesh = plsc.VectorSubcoreMesh(core_axis_name="c", subcore_axis_name="s")
_sc_params = pltpu.CompilerParams(use_tc_tiling_on_sc=False)


def _z16():
    return jnp.zeros((16,), jnp.float32)


@functools.partial(
    pl.kernel,
    mesh=_mesh,
    compiler_params=_sc_params,
    out_type=(jax.ShapeDtypeStruct((_NC, _ACC, _CW), jnp.float32),
              jax.ShapeDtypeStruct((_NC, _ACC, _CW), jnp.float32)),
    scratch_types=[
        pltpu.VMEM((_CPW, _CHUNK), jnp.int32),
        pltpu.VMEM((_CPW, _CHUNK), jnp.int32),
        pltpu.VMEM((_CHUNK, _CW), jnp.float32),
        pltpu.VMEM((_CHUNK, _CW), jnp.float32),
        pltpu.VMEM_SHARED((_ACC, _CW), jnp.float32),
        pltpu.VMEM_SHARED((_ACC, _CW), jnp.float32),
    ],
)
def _sc_counts(dst_hbm, src_hbm, deg_out, cnt_out, didx, sidx, ones_v, zbuf,
               dacc, cacc):
    cid = lax.axis_index("c")
    sid = lax.axis_index("s")
    wid = cid * _NS + sid

    def fill(i, carry):
        zbuf[i, :] = _z16()
        ones_v[i, :] = jnp.ones((16,), jnp.float32)
        return carry
    lax.fori_loop(0, _CHUNK, fill, 0)

    zb = sid * (_ACC // _NS)

    def zcp(i, carry):
        pltpu.sync_copy(zbuf, dacc.at[pl.ds(zb + i * _CHUNK, _CHUNK)])
        pltpu.sync_copy(zbuf, cacc.at[pl.ds(zb + i * _CHUNK, _CHUNK)])
        return carry
    lax.fori_loop(0, _ZPC, zcp, 0)
    plsc.subcore_barrier()

    pltpu.sync_copy(dst_hbm.at[pl.ds(wid * _CPW, _CPW)], didx)
    pltpu.sync_copy(src_hbm.at[pl.ds(wid * _CPW, _CPW)], sidx)

    def body(c, carry):
        pltpu.sync_copy(ones_v, dacc.at[didx.at[c]], add=True)
        pltpu.sync_copy(ones_v, cacc.at[sidx.at[c]], add=True)
        return carry
    lax.fori_loop(0, _CPW, body, 0)
    plsc.subcore_barrier()

    db = sid * _DPS
    pltpu.sync_copy(dacc.at[pl.ds(db, _DPS)], deg_out.at[cid, pl.ds(db, _DPS)])
    pltpu.sync_copy(cacc.at[pl.ds(db, _DPS)], cnt_out.at[cid, pl.ds(db, _DPS)])


_CPW0 = 160
_CPW1 = 0
_C1BASE = _NS * _CPW0


_NBUF = 4


def _pipe_gather_scatter(tab_hbm, gidx, sidx, rows, sems, acc, cpw):
    for j in range(_NBUF):
        pltpu.async_copy(tab_hbm.at[gidx.at[j]], rows[j], sems[j])

    def body(t, carry):
        for j in range(_NBUF):
            c = _NBUF * t + j
            pltpu.make_async_copy(tab_hbm.at[gidx.at[c]], rows[j],
                                  sems[j]).wait()
            pltpu.sync_copy(rows[j], acc.at[sidx.at[c]], add=True)

            @pl.when(c + _NBUF < cpw)
            def _():
                pltpu.async_copy(tab_hbm.at[gidx.at[c + _NBUF]], rows[j],
                                 sems[j])
        return carry
    lax.fori_loop(0, cpw // _NBUF, body, 0)


@functools.partial(
    pl.kernel,
    mesh=_mesh,
    compiler_params=_sc_params,
    out_type=jax.ShapeDtypeStruct((_NC, _ACC, _H), jnp.float32),
    scratch_types=[
        pltpu.VMEM((_CPW0, _CHUNK), jnp.int32),
        pltpu.VMEM((_CPW0, _CHUNK), jnp.int32),
        pltpu.VMEM((max(_CPW1, 8), _CHUNK), jnp.int32),
        pltpu.VMEM((max(_CPW1, 8), _CHUNK), jnp.int32),
        pltpu.VMEM((_NBUF, _CHUNK, _H), jnp.float32),
        pltpu.VMEM_SHARED((_ACC, _H), jnp.float32),
    ] + [pltpu.SemaphoreType.DMA] * _NBUF,
)
def _sc_rowpass(tab_hbm, gidx_hbm, sidx_hbm, out_hbm, gidx, sidx, gidx1,
                sidx1, rowsb, acc, *sems):
    cid = lax.axis_index("c")
    sid = lax.axis_index("s")

    rows = [rowsb.at[j] for j in range(_NBUF)]

    def zfill(i, carry):
        for j in range(_H // 16):
            rowsb[0, i, pl.ds(j * 16, 16)] = _z16()
        return carry
    lax.fori_loop(0, _CHUNK, zfill, 0)

    zb = sid * (_ACC // _NS)

    def zcp(i, carry):
        pltpu.sync_copy(rows[0], acc.at[pl.ds(zb + i * _CHUNK, _CHUNK)])
        return carry
    lax.fori_loop(0, _ZPC, zcp, 0)
    plsc.subcore_barrier()

    @pl.when(cid == 0)
    def _core0():
        hb = sid * _CPW0
        pltpu.sync_copy(gidx_hbm.at[pl.ds(hb, _CPW0)], gidx)
        pltpu.sync_copy(sidx_hbm.at[pl.ds(hb, _CPW0)], sidx)
        _pipe_gather_scatter(tab_hbm, gidx, sidx, rows, sems, acc, _CPW0)

    if _CPW1 > 0:
        @pl.when(cid == 1)
        def _core1():
            hb = _C1BASE + sid * _CPW1
            pltpu.sync_copy(gidx_hbm.at[pl.ds(hb, _CPW1)], gidx1)
            pltpu.sync_copy(sidx_hbm.at[pl.ds(hb, _CPW1)], sidx1)
            _pipe_gather_scatter(tab_hbm, gidx1, sidx1, rows, sems, acc,
                                 _CPW1)

    plsc.subcore_barrier()

    db = sid * _DPS
    pltpu.sync_copy(acc.at[pl.ds(db, _DPS)], out_hbm.at[cid, pl.ds(db, _DPS)])


_CPW2 = _EPAD // _CHUNK // _NS


@functools.partial(
    pl.kernel,
    mesh=_mesh,
    compiler_params=_sc_params,
    out_type=(jax.ShapeDtypeStruct((_ACC, _H), jnp.float32),
              jax.ShapeDtypeStruct((_ACC, _H), jnp.float32)),
    scratch_types=[
        pltpu.VMEM((_CPW2, _CHUNK), jnp.int32),
        pltpu.VMEM((_CPW2, _CHUNK), jnp.int32),
        pltpu.VMEM((_NBUF, _CHUNK, _H), jnp.float32),
        pltpu.VMEM_SHARED((_ACC, _H), jnp.float32),
    ] + [pltpu.SemaphoreType.DMA] * _NBUF,
)
def _sc_edgetime(eenc_hbm, tenc_hbm, sidx_hbm, tidx_hbm, dep_hbm, agge_out,
                 aggt_out, sidx, tidx, rowsb, acc, *sems):
    cid = lax.axis_index("c")
    sid = lax.axis_index("s")

    rows = [rowsb.at[j] for j in range(_NBUF)]

    def zfill(i, carry):
        for j in range(_H // 16):
            rowsb[0, i, pl.ds(j * 16, 16)] = _z16()
        return carry
    lax.fori_loop(0, _CHUNK, zfill, 0)

    zb = sid * (_ACC // _NS)

    def zcp(i, carry):
        pltpu.sync_copy(rows[0], acc.at[pl.ds(zb + i * _CHUNK, _CHUNK)])
        return carry
    lax.fori_loop(0, _ZPC, zcp, 0)
    plsc.subcore_barrier()

    pltpu.sync_copy(sidx_hbm.at[pl.ds(sid * _CPW2, _CPW2)], sidx)

    @pl.when(cid == 1)
    def _edge_job():
        emax = eenc_hbm.shape[0] - _CHUNK
        ebase = sid * (_CPW2 * _CHUNK)

        def rd(c):
            return jnp.minimum(ebase + c * _CHUNK, emax)
        for j in range(_NBUF):
            pltpu.async_copy(eenc_hbm.at[pl.ds(rd(j), _CHUNK)], rows[j],
                             sems[j])

        def body(t, carry):
            for j in range(_NBUF):
                c = _NBUF * t + j
                pltpu.make_async_copy(eenc_hbm.at[pl.ds(rd(c), _CHUNK)],
                                      rows[j], sems[j]).wait()
                pltpu.sync_copy(rows[j], acc.at[sidx.at[c]], add=True)

                @pl.when(c + _NBUF < _CPW2)
                def _():
                    pltpu.async_copy(
                        eenc_hbm.at[pl.ds(rd(c + _NBUF), _CHUNK)], rows[j],
                        sems[j])
            return carry
        lax.fori_loop(0, _CPW2 // _NBUF, body, 0)

    @pl.when(cid == 0)
    def _time_job():
        pltpu.sync_copy(tidx_hbm.at[pl.ds(sid * _CPW2, _CPW2)], tidx)
        _pipe_gather_scatter(tenc_hbm, tidx, sidx, rows, sems, acc, _CPW2)

    plsc.subcore_barrier()

    db = sid * _DPS

    @pl.when(cid == 1)
    def _dump_e():
        pltpu.sync_copy(acc.at[pl.ds(db, _DPS)], agge_out.at[pl.ds(db, _DPS)])

    @pl.when(cid == 0)
    def _dump_t():
        pltpu.sync_copy(acc.at[pl.ds(db, _DPS)], aggt_out.at[pl.ds(db, _DPS)])


def _enc_body(x_ref, wn_ref, bn_ref, wg1_ref, wres_ref, xe_ref, xw1_ref,
              res_ref):
    xe = jnp.maximum(jnp.dot(x_ref[...], wn_ref[...],
                             preferred_element_type=jnp.float32) + bn_ref[...],
                     0.0)
    xe_ref[...] = xe
    xw1_ref[...] = jnp.dot(xe, wg1_ref[...], preferred_element_type=jnp.float32)
    res_ref[...] = jnp.dot(xe, wres_ref[...], preferred_element_type=jnp.float32)


def _k_enc(x, W_node, b_node, W_g1, W_res):
    n, d = x.shape
    grid = (n // _BN,)
    return pl.pallas_call(
        _enc_body,
        grid=grid,
        in_specs=[
            pl.BlockSpec((_BN, d), lambda i: (i, 0)),
            pl.BlockSpec((d, _H), lambda i: (0, 0)),
            pl.BlockSpec((1, _H), lambda i: (0, 0)),
            pl.BlockSpec((_H, _H), lambda i: (0, 0)),
            pl.BlockSpec((_H, _H), lambda i: (0, 0)),
        ],
        out_specs=[pl.BlockSpec((_BN, _H), lambda i: (i, 0))] * 3,
        out_shape=[jax.ShapeDtypeStruct((n, _H), jnp.float32)] * 3,
    )(x, W_node, b_node, W_g1, W_res)


def _edge_mm_body(aT_ref, w_ref, b_ref, o_ref):
    o_ref[...] = jnp.maximum(
        lax.dot_general(aT_ref[...], w_ref[...], (((0,), (0,)), ((), ())),
                        preferred_element_type=jnp.float32) + b_ref[...], 0.0)


def _k_edge_mm(aT, W, b, blk):
    d, n = aT.shape
    grid = (n // blk,)
    return pl.pallas_call(
        _edge_mm_body,
        grid=grid,
        in_specs=[
            pl.BlockSpec((d, blk), lambda i: (0, i)),
            pl.BlockSpec((d, _H), lambda i: (0, 0)),
            pl.BlockSpec((1, _H), lambda i: (0, 0)),
        ],
        out_specs=pl.BlockSpec((blk, _H), lambda i: (i, 0)),
        out_shape=jax.ShapeDtypeStruct((n, _H), jnp.float32),
    )(aT, W, b)


def _mm_relu_body(a_ref, w_ref, b_ref, o_ref):
    o_ref[...] = jnp.maximum(
        jnp.dot(a_ref[...], w_ref[...], preferred_element_type=jnp.float32)
        + b_ref[...], 0.0)


def _k_mm_relu(a, W, b, blk):
    n, d = a.shape
    grid = (n // blk,)
    return pl.pallas_call(
        _mm_relu_body,
        grid=grid,
        in_specs=[
            pl.BlockSpec((blk, d), lambda i: (i, 0)),
            pl.BlockSpec((d, _H), lambda i: (0, 0)),
            pl.BlockSpec((1, _H), lambda i: (0, 0)),
        ],
        out_specs=pl.BlockSpec((blk, _H), lambda i: (i, 0)),
        out_shape=jax.ShapeDtypeStruct((n, _H), jnp.float32),
    )(a, W, b)


def _prep_body(degp_ref, xw1_ref, xws1_ref):
    deg = degp_ref[0][:, :1] + degp_ref[1][:, :1] + 1.0
    xws1_ref[...] = xw1_ref[...] * lax.rsqrt(deg)


def _k_prep(degp, xw1):
    grid = (_N // _BN,)
    return pl.pallas_call(
        _prep_body,
        grid=grid,
        in_specs=[
            pl.BlockSpec((_NC, _BN, _CW), lambda i: (0, i, 0)),
            pl.BlockSpec((_BN, _H), lambda i: (i, 0)),
        ],
        out_specs=pl.BlockSpec((_BN, _H), lambda i: (i, 0)),
        out_shape=jax.ShapeDtypeStruct((_N, _H), jnp.float32),
    )(degp, xw1)


def _mid1_body(p_ref, degp_ref, xw1_ref, wg2_ref, bg1_ref, xw2_ref, xws2_ref):
    deg = degp_ref[0][:, :1] + degp_ref[1][:, :1] + 1.0
    dis = lax.rsqrt(deg)
    x1 = jnp.maximum((p_ref[0] + p_ref[1]) * dis + xw1_ref[...] / deg
                     + bg1_ref[...], 0.0)
    xw2 = jnp.dot(x1, wg2_ref[...], preferred_element_type=jnp.float32)
    xw2_ref[...] = xw2
    xws2_ref[...] = xw2 * dis


def _k_mid1(out1p, degp, xw1, W_g2, b_g1):
    grid = (_N // _BN,)
    return pl.pallas_call(
        _mid1_body,
        grid=grid,
        in_specs=[
            pl.BlockSpec((_NC, _BN, _H), lambda i: (0, i, 0)),
            pl.BlockSpec((_NC, _BN, _CW), lambda i: (0, i, 0)),
            pl.BlockSpec((_BN, _H), lambda i: (i, 0)),
            pl.BlockSpec((_H, _H), lambda i: (0, 0)),
            pl.BlockSpec((1, _H), lambda i: (0, 0)),
        ],
        out_specs=[pl.BlockSpec((_BN, _H), lambda i: (i, 0))] * 2,
        out_shape=[jax.ShapeDtypeStruct((_N, _H), jnp.float32)] * 2,
    )(out1p, degp, xw1, W_g2, b_g1)


def _mid2_body(p_ref, degp_ref, xw2_ref, res_ref, bg2_ref, bres_ref, x2_ref):
    deg = degp_ref[0][:, :1] + degp_ref[1][:, :1] + 1.0
    dis = lax.rsqrt(deg)
    x2_ref[...] = jnp.maximum(
        (p_ref[0] + p_ref[1]) * dis + xw2_ref[...] / deg + bg2_ref[...]
        + res_ref[...] + bres_ref[...], 0.0)


def _k_mid2(out2p, degp, xw2, res, b_g2, b_res):
    grid = (_N // _BN,)
    return pl.pallas_call(
        _mid2_body,
        grid=grid,
        in_specs=[
            pl.BlockSpec((_NC, _BN, _H), lambda i: (0, i, 0)),
            pl.BlockSpec((_NC, _BN, _CW), lambda i: (0, i, 0)),
            pl.BlockSpec((_BN, _H), lambda i: (i, 0)),
            pl.BlockSpec((_BN, _H), lambda i: (i, 0)),
            pl.BlockSpec((1, _H), lambda i: (0, 0)),
            pl.BlockSpec((1, _H), lambda i: (0, 0)),
        ],
        out_specs=pl.BlockSpec((_BN, _H), lambda i: (i, 0)),
        out_shape=jax.ShapeDtypeStruct((_N, _H), jnp.float32),
    )(out2p, degp, xw2, res, b_g2, b_res)


def _final_body(aggdp_ref, aggep_ref, aggtp_ref, cntp_ref, x2_ref, xe_ref,
                wc1_ref, bc1_ref, wc2_ref, bc2_ref, o_ref):
    scnt = cntp_ref[0][:, :1] + cntp_ref[1][:, :1]
    c = jnp.maximum(scnt, 1.0)
    inv = 1.0 / c
    nf0 = scnt * x2_ref[...] * inv
    nf1 = (aggdp_ref[0] + aggdp_ref[1]) * inv
    nf2 = aggep_ref[...] * inv
    nf3 = aggtp_ref[...] * inv
    s = jnp.sum(jnp.abs(nf0) + jnp.abs(nf1) + jnp.abs(nf2) + jnp.abs(nf3),
                axis=1, keepdims=True)
    m = s < 1e-6
    xe = xe_ref[...]
    nf0 = jnp.where(m, xe, nf0)
    nf1 = jnp.where(m, xe, nf1)
    nf2 = jnp.where(m, xe, nf2)
    nf3 = jnp.where(m, xe, nf3)
    h = (jnp.dot(nf0, wc1_ref[0], preferred_element_type=jnp.float32)
         + jnp.dot(nf1, wc1_ref[1], preferred_element_type=jnp.float32)
         + jnp.dot(nf2, wc1_ref[2], preferred_element_type=jnp.float32)
         + jnp.dot(nf3, wc1_ref[3], preferred_element_type=jnp.float32)
         + bc1_ref[...])
    h = jnp.maximum(h, 0.0)
    o_ref[...] = jnp.dot(h, wc2_ref[...],
                         preferred_element_type=jnp.float32) + bc2_ref[...]


def _k_final(aggdp, aggep, aggtp, cntp, x2, xe, Wc1r, b_c1, Wc2p, bc2p):
    grid = (_N // _BN,)
    return pl.pallas_call(
        _final_body,
        grid=grid,
        in_specs=[
            pl.BlockSpec((_NC, _BN, _H), lambda i: (0, i, 0)),
            pl.BlockSpec((_BN, _H), lambda i: (i, 0)),
            pl.BlockSpec((_BN, _H), lambda i: (i, 0)),
            pl.BlockSpec((_NC, _BN, _CW), lambda i: (0, i, 0)),
            pl.BlockSpec((_BN, _H), lambda i: (i, 0)),
            pl.BlockSpec((_BN, _H), lambda i: (i, 0)),
            pl.BlockSpec((4, _H, _H), lambda i: (0, 0, 0)),
            pl.BlockSpec((1, _H), lambda i: (0, 0)),
            pl.BlockSpec((_H, 128), lambda i: (0, 0)),
            pl.BlockSpec((1, 128), lambda i: (0, 0)),
        ],
        out_specs=pl.BlockSpec((_BN, 128), lambda i: (i, 0)),
        out_shape=jax.ShapeDtypeStruct((_N, 128), jnp.float32),
    )(aggdp, aggep, aggtp, cntp, x2, xe, Wc1r, b_c1, Wc2p, bc2p)


def kernel(x, edge_index, edge_attr, timestamps, W_node, b_node, W_edge,
           b_edge, time_table, W_time, b_time, W_g1, b_g1, W_g2, b_g2, W_res,
           b_res, W_c1, b_c1, W_c2, b_c2):
    E = edge_index.shape[1]
    pad = _EPAD - E
    src = edge_index[0]
    dst = edge_index[1]
    zpad = jnp.zeros((pad,), jnp.int32)
    spad = jnp.full((pad,), _SINK, jnp.int32)
    srcZ = jnp.concatenate([src, zpad]).reshape(-1, _CHUNK)
    srcS = jnp.concatenate([src, spad]).reshape(-1, _CHUNK)
    dstZ = jnp.concatenate([dst, zpad]).reshape(-1, _CHUNK)
    dstS = jnp.concatenate([dst, spad]).reshape(-1, _CHUNK)
    ts = jnp.clip(timestamps, 0, time_table.shape[0] - 1).astype(jnp.int32)
    tsZ = jnp.concatenate([ts, zpad]).reshape(-1, _CHUNK)
    def b2(v):
        return v.reshape(1, -1)

    xe, xw1, res = _k_enc(x, W_node, b2(b_node), W_g1, W_res)
    tenc = _k_mm_relu(time_table, W_time, b2(b_time), time_table.shape[0])
    eenc = _k_edge_mm(edge_attr.T, W_edge, b2(b_edge), 12800)

    degp, cntp = _sc_counts(dstS, srcS)

    xws1 = _k_prep(degp, xw1)
    out1p = _sc_rowpass(xws1, srcZ, dstS)
    xw2, xws2 = _k_mid1(out1p, degp, xw1, W_g2, b2(b_g1))

    out2p = _sc_rowpass(xws2, srcZ, dstS)
    x2 = _k_mid2(out2p, degp, xw2, res, b2(b_g2), b2(b_res))

    aggdp = _sc_rowpass(x2, dstZ, srcS)
    aggep, aggtp = _sc_edgetime(eenc, tenc, srcS, tsZ, aggdp)

    Wc1r = W_c1.reshape(4, _H, _H)
    nout = W_c2.shape[1]
    Wc2p = jnp.zeros((_H, 128), jnp.float32).at[:, :nout].set(W_c2)
    bc2p = jnp.zeros((1, 128), jnp.float32).at[0, :nout].set(b_c2)
    outp = _k_final(aggdp, aggep, aggtp, cntp, x2, xe, Wc1r, b2(b_c1), Wc2p,
                    bc2p)
    return outp[:, :nout]


# speedup vs baseline: 1.0730x; 1.0730x over previous
import functools
import jax
import jax.numpy as jnp
from jax import lax
from jax.experimental import pallas as pl
from jax.experimental.pallas import tpu as pltpu
from jax.experimental.pallas import tpu_sc as plsc

_N = 10000
_H = 64
_NC = 2
_NS = 16
_NW = _NC * _NS
_CHUNK = 128
_CPW = 80
_EPAD = _NW * _CPW * _CHUNK
_ACC = 10240
_SINK = _N
_ZPC = _ACC // _NS // _CHUNK
_DPS = _ACC // _NS
_CW = 16
_BN = 2000

_mesh = plsc.VectorSubcoreMesh(core_axis_name="c", subcore_axis_name="s")
_sc_params = pltpu.CompilerParams(use_tc_tiling_on_sc=False)
_sc_cost = pl.CostEstimate(flops=0, bytes_accessed=400_000_000,
                           transcendentals=0)


def _z16():
    return jnp.zeros((16,), jnp.float32)


@functools.partial(
    pl.kernel,
    mesh=_mesh,
    compiler_params=_sc_params,
    cost_estimate=_sc_cost,
    out_type=(jax.ShapeDtypeStruct((_NC, _ACC, _CW), jnp.float32),
              jax.ShapeDtypeStruct((_NC, _ACC, _CW), jnp.float32)),
    scratch_types=[
        pltpu.VMEM((_CPW, _CHUNK), jnp.int32),
        pltpu.VMEM((_CPW, _CHUNK), jnp.int32),
        pltpu.VMEM((_CHUNK, _CW), jnp.float32),
        pltpu.VMEM((_CHUNK, _CW), jnp.float32),
        pltpu.VMEM_SHARED((_ACC, _CW), jnp.float32),
        pltpu.VMEM_SHARED((_ACC, _CW), jnp.float32),
    ],
)
def _sc_counts(dst_hbm, src_hbm, deg_out, cnt_out, didx, sidx, ones_v, zbuf,
               dacc, cacc):
    cid = lax.axis_index("c")
    sid = lax.axis_index("s")
    wid = cid * _NS + sid

    def fill(i, carry):
        zbuf[i, :] = _z16()
        ones_v[i, :] = jnp.ones((16,), jnp.float32)
        return carry
    lax.fori_loop(0, _CHUNK, fill, 0)

    zb = sid * (_ACC // _NS)

    def zcp(i, carry):
        pltpu.sync_copy(zbuf, dacc.at[pl.ds(zb + i * _CHUNK, _CHUNK)])
        pltpu.sync_copy(zbuf, cacc.at[pl.ds(zb + i * _CHUNK, _CHUNK)])
        return carry
    lax.fori_loop(0, _ZPC, zcp, 0)
    plsc.subcore_barrier()

    pltpu.sync_copy(dst_hbm.at[pl.ds(wid * _CPW, _CPW)], didx)
    pltpu.sync_copy(src_hbm.at[pl.ds(wid * _CPW, _CPW)], sidx)

    def body(c, carry):
        pltpu.sync_copy(ones_v, dacc.at[didx.at[c]], add=True)
        pltpu.sync_copy(ones_v, cacc.at[sidx.at[c]], add=True)
        return carry
    lax.fori_loop(0, _CPW, body, 0)
    plsc.subcore_barrier()

    db = sid * _DPS
    pltpu.sync_copy(dacc.at[pl.ds(db, _DPS)], deg_out.at[cid, pl.ds(db, _DPS)])
    pltpu.sync_copy(cacc.at[pl.ds(db, _DPS)], cnt_out.at[cid, pl.ds(db, _DPS)])


_CPW0 = 120
_CPW1 = 40
_C1BASE = _NS * _CPW0


_NBUF = 4


def _pipe_gather_scatter(tab_hbm, gidx, sidx, rows, sems, acc, cpw):
    for j in range(_NBUF):
        pltpu.async_copy(tab_hbm.at[gidx.at[j]], rows[j], sems[j])

    def body(t, carry):
        for j in range(_NBUF):
            c = _NBUF * t + j
            pltpu.make_async_copy(tab_hbm.at[gidx.at[c]], rows[j],
                                  sems[j]).wait()
            pltpu.sync_copy(rows[j], acc.at[sidx.at[c]], add=True)

            @pl.when(c + _NBUF < cpw)
            def _():
                pltpu.async_copy(tab_hbm.at[gidx.at[c + _NBUF]], rows[j],
                                 sems[j])
        return carry
    lax.fori_loop(0, cpw // _NBUF, body, 0)


@functools.partial(
    pl.kernel,
    mesh=_mesh,
    compiler_params=_sc_params,
    cost_estimate=_sc_cost,
    out_type=jax.ShapeDtypeStruct((_NC, _ACC, _H), jnp.float32),
    scratch_types=[
        pltpu.VMEM((_CPW0, _CHUNK), jnp.int32),
        pltpu.VMEM((_CPW0, _CHUNK), jnp.int32),
        pltpu.VMEM((max(_CPW1, 8), _CHUNK), jnp.int32),
        pltpu.VMEM((max(_CPW1, 8), _CHUNK), jnp.int32),
        pltpu.VMEM((_NBUF, _CHUNK, _H), jnp.float32),
        pltpu.VMEM_SHARED((_ACC, _H), jnp.float32),
    ] + [pltpu.SemaphoreType.DMA] * _NBUF,
)
def _sc_rowpass(tab_hbm, gidx_hbm, sidx_hbm, out_hbm, gidx, sidx, gidx1,
                sidx1, rowsb, acc, *sems):
    cid = lax.axis_index("c")
    sid = lax.axis_index("s")

    rows = [rowsb.at[j] for j in range(_NBUF)]

    def zfill(i, carry):
        for j in range(_H // 16):
            rowsb[0, i, pl.ds(j * 16, 16)] = _z16()
        return carry
    lax.fori_loop(0, _CHUNK, zfill, 0)

    zb = sid * (_ACC // _NS)

    def zcp(i, carry):
        pltpu.sync_copy(rows[0], acc.at[pl.ds(zb + i * _CHUNK, _CHUNK)])
        return carry
    lax.fori_loop(0, _ZPC, zcp, 0)
    plsc.subcore_barrier()

    @pl.when(cid == 0)
    def _core0():
        hb = sid * _CPW0
        pltpu.sync_copy(gidx_hbm.at[pl.ds(hb, _CPW0)], gidx)
        pltpu.sync_copy(sidx_hbm.at[pl.ds(hb, _CPW0)], sidx)
        _pipe_gather_scatter(tab_hbm, gidx, sidx, rows, sems, acc, _CPW0)

    if _CPW1 > 0:
        @pl.when(cid == 1)
        def _core1():
            hb = _C1BASE + sid * _CPW1
            pltpu.sync_copy(gidx_hbm.at[pl.ds(hb, _CPW1)], gidx1)
            pltpu.sync_copy(sidx_hbm.at[pl.ds(hb, _CPW1)], sidx1)
            _pipe_gather_scatter(tab_hbm, gidx1, sidx1, rows, sems, acc,
                                 _CPW1)

    plsc.subcore_barrier()

    db = sid * _DPS
    pltpu.sync_copy(acc.at[pl.ds(db, _DPS)], out_hbm.at[cid, pl.ds(db, _DPS)])


_CPW2 = _EPAD // _CHUNK // _NS


@functools.partial(
    pl.kernel,
    mesh=_mesh,
    compiler_params=_sc_params,
    cost_estimate=_sc_cost,
    out_type=(jax.ShapeDtypeStruct((_ACC, _H), jnp.float32),
              jax.ShapeDtypeStruct((_ACC, _H), jnp.float32)),
    scratch_types=[
        pltpu.VMEM((_CPW2, _CHUNK), jnp.int32),
        pltpu.VMEM((_CPW2, _CHUNK), jnp.int32),
        pltpu.VMEM((_NBUF, _CHUNK, _H), jnp.float32),
        pltpu.VMEM_SHARED((_ACC, _H), jnp.float32),
    ] + [pltpu.SemaphoreType.DMA] * _NBUF,
)
def _sc_edgetime(eenc_hbm, tenc_hbm, sidx_hbm, tidx_hbm, dep_hbm, agge_out,
                 aggt_out, sidx, tidx, rowsb, acc, *sems):
    cid = lax.axis_index("c")
    sid = lax.axis_index("s")

    rows = [rowsb.at[j] for j in range(_NBUF)]

    def zfill(i, carry):
        for j in range(_H // 16):
            rowsb[0, i, pl.ds(j * 16, 16)] = _z16()
        return carry
    lax.fori_loop(0, _CHUNK, zfill, 0)

    zb = sid * (_ACC // _NS)

    def zcp(i, carry):
        pltpu.sync_copy(rows[0], acc.at[pl.ds(zb + i * _CHUNK, _CHUNK)])
        return carry
    lax.fori_loop(0, _ZPC, zcp, 0)
    plsc.subcore_barrier()

    pltpu.sync_copy(sidx_hbm.at[pl.ds(sid * _CPW2, _CPW2)], sidx)

    @pl.when(cid == 1)
    def _edge_job():
        emax = eenc_hbm.shape[0] - _CHUNK
        ebase = sid * (_CPW2 * _CHUNK)

        def rd(c):
            return jnp.minimum(ebase + c * _CHUNK, emax)
        for j in range(_NBUF):
            pltpu.async_copy(eenc_hbm.at[pl.ds(rd(j), _CHUNK)], rows[j],
                             sems[j])

        def body(t, carry):
            for j in range(_NBUF):
                c = _NBUF * t + j
                pltpu.make_async_copy(eenc_hbm.at[pl.ds(rd(c), _CHUNK)],
                                      rows[j], sems[j]).wait()
                pltpu.sync_copy(rows[j], acc.at[sidx.at[c]], add=True)

                @pl.when(c + _NBUF < _CPW2)
                def _():
                    pltpu.async_copy(
                        eenc_hbm.at[pl.ds(rd(c + _NBUF), _CHUNK)], rows[j],
                        sems[j])
            return carry
        lax.fori_loop(0, _CPW2 // _NBUF, body, 0)

    @pl.when(cid == 0)
    def _time_job():
        pltpu.sync_copy(tidx_hbm.at[pl.ds(sid * _CPW2, _CPW2)], tidx)
        _pipe_gather_scatter(tenc_hbm, tidx, sidx, rows, sems, acc, _CPW2)

    plsc.subcore_barrier()

    db = sid * _DPS

    @pl.when(cid == 1)
    def _dump_e():
        pltpu.sync_copy(acc.at[pl.ds(db, _DPS)], agge_out.at[pl.ds(db, _DPS)])

    @pl.when(cid == 0)
    def _dump_t():
        pltpu.sync_copy(acc.at[pl.ds(db, _DPS)], aggt_out.at[pl.ds(db, _DPS)])


def _enc_body(x_ref, wn_ref, bn_ref, wg1_ref, wres_ref, xe_ref, xw1_ref,
              res_ref):
    xe = jnp.maximum(jnp.dot(x_ref[...], wn_ref[...],
                             preferred_element_type=jnp.float32) + bn_ref[...],
                     0.0)
    xe_ref[...] = xe
    xw1_ref[...] = jnp.dot(xe, wg1_ref[...], preferred_element_type=jnp.float32)
    res_ref[...] = jnp.dot(xe, wres_ref[...], preferred_element_type=jnp.float32)


def _k_enc(x, W_node, b_node, W_g1, W_res):
    n, d = x.shape
    grid = (n // _BN,)
    return pl.pallas_call(
        _enc_body,
        grid=grid,
        in_specs=[
            pl.BlockSpec((_BN, d), lambda i: (i, 0)),
            pl.BlockSpec((d, _H), lambda i: (0, 0)),
            pl.BlockSpec((1, _H), lambda i: (0, 0)),
            pl.BlockSpec((_H, _H), lambda i: (0, 0)),
            pl.BlockSpec((_H, _H), lambda i: (0, 0)),
        ],
        out_specs=[pl.BlockSpec((_BN, _H), lambda i: (i, 0))] * 3,
        out_shape=[jax.ShapeDtypeStruct((n, _H), jnp.float32)] * 3,
    )(x, W_node, b_node, W_g1, W_res)


def _edge_mm_body(aT_ref, w_ref, b_ref, o_ref):
    o_ref[...] = jnp.maximum(
        lax.dot_general(aT_ref[...], w_ref[...], (((0,), (0,)), ((), ())),
                        preferred_element_type=jnp.float32) + b_ref[...], 0.0)


def _k_edge_mm(aT, W, b, blk):
    d, n = aT.shape
    grid = (n // blk,)
    return pl.pallas_call(
        _edge_mm_body,
        grid=grid,
        in_specs=[
            pl.BlockSpec((d, blk), lambda i: (0, i)),
            pl.BlockSpec((d, _H), lambda i: (0, 0)),
            pl.BlockSpec((1, _H), lambda i: (0, 0)),
        ],
        out_specs=pl.BlockSpec((blk, _H), lambda i: (i, 0)),
        out_shape=jax.ShapeDtypeStruct((n, _H), jnp.float32),
    )(aT, W, b)


def _mm_relu_body(a_ref, w_ref, b_ref, o_ref):
    o_ref[...] = jnp.maximum(
        jnp.dot(a_ref[...], w_ref[...], preferred_element_type=jnp.float32)
        + b_ref[...], 0.0)


def _k_mm_relu(a, W, b, blk):
    n, d = a.shape
    grid = (n // blk,)
    return pl.pallas_call(
        _mm_relu_body,
        grid=grid,
        in_specs=[
            pl.BlockSpec((blk, d), lambda i: (i, 0)),
            pl.BlockSpec((d, _H), lambda i: (0, 0)),
            pl.BlockSpec((1, _H), lambda i: (0, 0)),
        ],
        out_specs=pl.BlockSpec((blk, _H), lambda i: (i, 0)),
        out_shape=jax.ShapeDtypeStruct((n, _H), jnp.float32),
    )(a, W, b)


def _prep_body(degp_ref, xw1_ref, xws1_ref):
    deg = degp_ref[0][:, :1] + degp_ref[1][:, :1] + 1.0
    xws1_ref[...] = xw1_ref[...] * lax.rsqrt(deg)


def _k_prep(degp, xw1):
    grid = (_N // _BN,)
    return pl.pallas_call(
        _prep_body,
        grid=grid,
        in_specs=[
            pl.BlockSpec((_NC, _BN, _CW), lambda i: (0, i, 0)),
            pl.BlockSpec((_BN, _H), lambda i: (i, 0)),
        ],
        out_specs=pl.BlockSpec((_BN, _H), lambda i: (i, 0)),
        out_shape=jax.ShapeDtypeStruct((_N, _H), jnp.float32),
    )(degp, xw1)


def _mid1_body(p_ref, degp_ref, xw1_ref, wg2_ref, bg1_ref, xw2_ref, xws2_ref):
    deg = degp_ref[0][:, :1] + degp_ref[1][:, :1] + 1.0
    dis = lax.rsqrt(deg)
    x1 = jnp.maximum((p_ref[0] + p_ref[1]) * dis + xw1_ref[...] / deg
                     + bg1_ref[...], 0.0)
    xw2 = jnp.dot(x1, wg2_ref[...], preferred_element_type=jnp.float32)
    xw2_ref[...] = xw2
    xws2_ref[...] = xw2 * dis


def _k_mid1(out1p, degp, xw1, W_g2, b_g1):
    grid = (_N // _BN,)
    return pl.pallas_call(
        _mid1_body,
        grid=grid,
        in_specs=[
            pl.BlockSpec((_NC, _BN, _H), lambda i: (0, i, 0)),
            pl.BlockSpec((_NC, _BN, _CW), lambda i: (0, i, 0)),
            pl.BlockSpec((_BN, _H), lambda i: (i, 0)),
            pl.BlockSpec((_H, _H), lambda i: (0, 0)),
            pl.BlockSpec((1, _H), lambda i: (0, 0)),
        ],
        out_specs=[pl.BlockSpec((_BN, _H), lambda i: (i, 0))] * 2,
        out_shape=[jax.ShapeDtypeStruct((_N, _H), jnp.float32)] * 2,
    )(out1p, degp, xw1, W_g2, b_g1)


def _mid2_body(p_ref, degp_ref, xw2_ref, res_ref, bg2_ref, bres_ref, x2_ref):
    deg = degp_ref[0][:, :1] + degp_ref[1][:, :1] + 1.0
    dis = lax.rsqrt(deg)
    x2_ref[...] = jnp.maximum(
        (p_ref[0] + p_ref[1]) * dis + xw2_ref[...] / deg + bg2_ref[...]
        + res_ref[...] + bres_ref[...], 0.0)


def _k_mid2(out2p, degp, xw2, res, b_g2, b_res):
    grid = (_N // _BN,)
    return pl.pallas_call(
        _mid2_body,
        grid=grid,
        in_specs=[
            pl.BlockSpec((_NC, _BN, _H), lambda i: (0, i, 0)),
            pl.BlockSpec((_NC, _BN, _CW), lambda i: (0, i, 0)),
            pl.BlockSpec((_BN, _H), lambda i: (i, 0)),
            pl.BlockSpec((_BN, _H), lambda i: (i, 0)),
            pl.BlockSpec((1, _H), lambda i: (0, 0)),
            pl.BlockSpec((1, _H), lambda i: (0, 0)),
        ],
        out_specs=pl.BlockSpec((_BN, _H), lambda i: (i, 0)),
        out_shape=jax.ShapeDtypeStruct((_N, _H), jnp.float32),
    )(out2p, degp, xw2, res, b_g2, b_res)


def _final_body(aggdp_ref, aggep_ref, aggtp_ref, cntp_ref, x2_ref, xe_ref,
                wc1_ref, bc1_ref, wc2_ref, bc2_ref, o_ref):
    scnt = cntp_ref[0][:, :1] + cntp_ref[1][:, :1]
    c = jnp.maximum(scnt, 1.0)
    inv = 1.0 / c
    nf0 = scnt * x2_ref[...] * inv
    nf1 = (aggdp_ref[0] + aggdp_ref[1]) * inv
    nf2 = aggep_ref[...] * inv
    nf3 = aggtp_ref[...] * inv
    s = jnp.sum(jnp.abs(nf0) + jnp.abs(nf1) + jnp.abs(nf2) + jnp.abs(nf3),
                axis=1, keepdims=True)
    m = s < 1e-6
    xe = xe_ref[...]
    nf0 = jnp.where(m, xe, nf0)
    nf1 = jnp.where(m, xe, nf1)
    nf2 = jnp.where(m, xe, nf2)
    nf3 = jnp.where(m, xe, nf3)
    h = (jnp.dot(nf0, wc1_ref[0], preferred_element_type=jnp.float32)
         + jnp.dot(nf1, wc1_ref[1], preferred_element_type=jnp.float32)
         + jnp.dot(nf2, wc1_ref[2], preferred_element_type=jnp.float32)
         + jnp.dot(nf3, wc1_ref[3], preferred_element_type=jnp.float32)
         + bc1_ref[...])
    h = jnp.maximum(h, 0.0)
    o_ref[...] = jnp.dot(h, wc2_ref[...],
                         preferred_element_type=jnp.float32) + bc2_ref[...]


def _k_final(aggdp, aggep, aggtp, cntp, x2, xe, Wc1r, b_c1, Wc2p, bc2p):
    grid = (_N // _BN,)
    return pl.pallas_call(
        _final_body,
        grid=grid,
        in_specs=[
            pl.BlockSpec((_NC, _BN, _H), lambda i: (0, i, 0)),
            pl.BlockSpec((_BN, _H), lambda i: (i, 0)),
            pl.BlockSpec((_BN, _H), lambda i: (i, 0)),
            pl.BlockSpec((_NC, _BN, _CW), lambda i: (0, i, 0)),
            pl.BlockSpec((_BN, _H), lambda i: (i, 0)),
            pl.BlockSpec((_BN, _H), lambda i: (i, 0)),
            pl.BlockSpec((4, _H, _H), lambda i: (0, 0, 0)),
            pl.BlockSpec((1, _H), lambda i: (0, 0)),
            pl.BlockSpec((_H, 128), lambda i: (0, 0)),
            pl.BlockSpec((1, 128), lambda i: (0, 0)),
        ],
        out_specs=pl.BlockSpec((_BN, 128), lambda i: (i, 0)),
        out_shape=jax.ShapeDtypeStruct((_N, 128), jnp.float32),
    )(aggdp, aggep, aggtp, cntp, x2, xe, Wc1r, b_c1, Wc2p, bc2p)


def kernel(x, edge_index, edge_attr, timestamps, W_node, b_node, W_edge,
           b_edge, time_table, W_time, b_time, W_g1, b_g1, W_g2, b_g2, W_res,
           b_res, W_c1, b_c1, W_c2, b_c2):
    E = edge_index.shape[1]
    pad = _EPAD - E
    src = edge_index[0]
    dst = edge_index[1]
    zpad = jnp.zeros((pad,), jnp.int32)
    spad = jnp.full((pad,), _SINK, jnp.int32)
    srcZ = jnp.concatenate([src, zpad]).reshape(-1, _CHUNK)
    srcS = jnp.concatenate([src, spad]).reshape(-1, _CHUNK)
    dstZ = jnp.concatenate([dst, zpad]).reshape(-1, _CHUNK)
    dstS = jnp.concatenate([dst, spad]).reshape(-1, _CHUNK)
    ts = jnp.clip(timestamps, 0, time_table.shape[0] - 1).astype(jnp.int32)
    tsZ = jnp.concatenate([ts, zpad]).reshape(-1, _CHUNK)
    def b2(v):
        return v.reshape(1, -1)

    xe, xw1, res = _k_enc(x, W_node, b2(b_node), W_g1, W_res)
    tenc = _k_mm_relu(time_table, W_time, b2(b_time), time_table.shape[0])
    eenc = _k_edge_mm(edge_attr.T, W_edge, b2(b_edge), 12800)

    degp, cntp = _sc_counts(dstS, srcS)

    xws1 = _k_prep(degp, xw1)
    out1p = _sc_rowpass(xws1, srcZ, dstS)
    xw2, xws2 = _k_mid1(out1p, degp, xw1, W_g2, b2(b_g1))

    out2p = _sc_rowpass(xws2, srcZ, dstS)
    x2 = _k_mid2(out2p, degp, xw2, res, b2(b_g2), b2(b_res))

    aggdp = _sc_rowpass(x2, dstZ, srcS)
    aggep, aggtp = _sc_edgetime(eenc, tenc, srcS, tsZ, aggdp)

    Wc1r = W_c1.reshape(4, _H, _H)
    nout = W_c2.shape[1]
    Wc2p = jnp.zeros((_H, 128), jnp.float32).at[:, :nout].set(W_c2)
    bc2p = jnp.zeros((1, 128), jnp.float32).at[0, :nout].set(b_c2)
    outp = _k_final(aggdp, aggep, aggtp, cntp, x2, xe, Wc1r, b2(b_c1), Wc2p,
                    bc2p)
    return outp[:, :nout]


# speedup vs baseline: 1.1869x; 1.1061x over previous
import functools
import jax
import jax.numpy as jnp
from jax import lax
from jax.experimental import pallas as pl
from jax.experimental.pallas import tpu as pltpu
from jax.experimental.pallas import tpu_sc as plsc

_N = 10000
_H = 64
_NC = 2
_NS = 16
_NW = _NC * _NS
_CHUNK = 128
_CPW = 80
_EPAD = _NW * _CPW * _CHUNK
_ACC = 10240
_SINK = _N
_ZPC = _ACC // _NS // _CHUNK
_DPS = _ACC // _NS
_CW = 16
_BN = 2000

_mesh = plsc.VectorSubcoreMesh(core_axis_name="c", subcore_axis_name="s")
_sc_params = pltpu.CompilerParams(use_tc_tiling_on_sc=False)
_sc_cost = pl.CostEstimate(flops=0, bytes_accessed=400_000_000,
                           transcendentals=0)


def _z16():
    return jnp.zeros((16,), jnp.float32)


@functools.partial(
    pl.kernel,
    mesh=_mesh,
    compiler_params=_sc_params,
    cost_estimate=_sc_cost,
    out_type=(jax.ShapeDtypeStruct((_NC, _ACC, _CW), jnp.float32),
              jax.ShapeDtypeStruct((_NC, _ACC, _CW), jnp.float32)),
    scratch_types=[
        pltpu.VMEM((_CPW, _CHUNK), jnp.int32),
        pltpu.VMEM((_CPW, _CHUNK), jnp.int32),
        pltpu.VMEM((_CHUNK, _CW), jnp.float32),
        pltpu.VMEM((_CHUNK, _CW), jnp.float32),
        pltpu.VMEM_SHARED((_ACC, _CW), jnp.float32),
        pltpu.VMEM_SHARED((_ACC, _CW), jnp.float32),
    ],
)
def _sc_counts(dst_hbm, src_hbm, deg_out, cnt_out, didx, sidx, ones_v, zbuf,
               dacc, cacc):
    cid = lax.axis_index("c")
    sid = lax.axis_index("s")
    wid = cid * _NS + sid

    def fill(i, carry):
        zbuf[i, :] = _z16()
        ones_v[i, :] = jnp.ones((16,), jnp.float32)
        return carry
    lax.fori_loop(0, _CHUNK, fill, 0)

    zb = sid * (_ACC // _NS)

    def zcp(i, carry):
        pltpu.sync_copy(zbuf, dacc.at[pl.ds(zb + i * _CHUNK, _CHUNK)])
        pltpu.sync_copy(zbuf, cacc.at[pl.ds(zb + i * _CHUNK, _CHUNK)])
        return carry
    lax.fori_loop(0, _ZPC, zcp, 0)
    plsc.subcore_barrier()

    pltpu.sync_copy(dst_hbm.at[pl.ds(wid * _CPW, _CPW)], didx)
    pltpu.sync_copy(src_hbm.at[pl.ds(wid * _CPW, _CPW)], sidx)

    def body(c, carry):
        pltpu.sync_copy(ones_v, dacc.at[didx.at[c]], add=True)
        pltpu.sync_copy(ones_v, cacc.at[sidx.at[c]], add=True)
        return carry
    lax.fori_loop(0, _CPW, body, 0)
    plsc.subcore_barrier()

    db = sid * _DPS
    pltpu.sync_copy(dacc.at[pl.ds(db, _DPS)], deg_out.at[cid, pl.ds(db, _DPS)])
    pltpu.sync_copy(cacc.at[pl.ds(db, _DPS)], cnt_out.at[cid, pl.ds(db, _DPS)])


_CPW0 = 120
_CPW1 = 40
_C1BASE = _NS * _CPW0


_NBUF = 4


def _pipe_gather_scatter(tab_hbm, gidx, sidx, rows, sems, acc, cpw):
    for j in range(_NBUF):
        pltpu.async_copy(tab_hbm.at[gidx.at[j]], rows[j], sems[j])

    def body(t, carry):
        for j in range(_NBUF):
            c = _NBUF * t + j
            pltpu.make_async_copy(tab_hbm.at[gidx.at[c]], rows[j],
                                  sems[j]).wait()
            pltpu.sync_copy(rows[j], acc.at[sidx.at[c]], add=True)

            @pl.when(c + _NBUF < cpw)
            def _():
                pltpu.async_copy(tab_hbm.at[gidx.at[c + _NBUF]], rows[j],
                                 sems[j])
        return carry
    lax.fori_loop(0, cpw // _NBUF, body, 0)


@functools.partial(
    pl.kernel,
    mesh=_mesh,
    compiler_params=_sc_params,
    cost_estimate=_sc_cost,
    out_type=jax.ShapeDtypeStruct((_NC, _ACC, _H), jnp.float32),
    scratch_types=[
        pltpu.VMEM((_CPW0, _CHUNK), jnp.int32),
        pltpu.VMEM((_CPW0, _CHUNK), jnp.int32),
        pltpu.VMEM((max(_CPW1, 8), _CHUNK), jnp.int32),
        pltpu.VMEM((max(_CPW1, 8), _CHUNK), jnp.int32),
        pltpu.VMEM((_NBUF, _CHUNK, _H), jnp.float32),
        pltpu.VMEM_SHARED((_ACC, _H), jnp.float32),
    ] + [pltpu.SemaphoreType.DMA] * _NBUF,
)
def _sc_rowpass(tab_hbm, gidx_hbm, sidx_hbm, out_hbm, gidx, sidx, gidx1,
                sidx1, rowsb, acc, *sems):
    cid = lax.axis_index("c")
    sid = lax.axis_index("s")

    rows = [rowsb.at[j] for j in range(_NBUF)]

    def zfill(i, carry):
        for j in range(_H // 16):
            rowsb[0, i, pl.ds(j * 16, 16)] = _z16()
        return carry
    lax.fori_loop(0, _CHUNK, zfill, 0)

    zb = sid * (_ACC // _NS)

    def zcp(i, carry):
        pltpu.sync_copy(rows[0], acc.at[pl.ds(zb + i * _CHUNK, _CHUNK)])
        return carry
    lax.fori_loop(0, _ZPC, zcp, 0)
    plsc.subcore_barrier()

    @pl.when(cid == 0)
    def _core0():
        hb = sid * _CPW0
        pltpu.sync_copy(gidx_hbm.at[pl.ds(hb, _CPW0)], gidx)
        pltpu.sync_copy(sidx_hbm.at[pl.ds(hb, _CPW0)], sidx)
        _pipe_gather_scatter(tab_hbm, gidx, sidx, rows, sems, acc, _CPW0)

    if _CPW1 > 0:
        @pl.when(cid == 1)
        def _core1():
            hb = _C1BASE + sid * _CPW1
            pltpu.sync_copy(gidx_hbm.at[pl.ds(hb, _CPW1)], gidx1)
            pltpu.sync_copy(sidx_hbm.at[pl.ds(hb, _CPW1)], sidx1)
            _pipe_gather_scatter(tab_hbm, gidx1, sidx1, rows, sems, acc,
                                 _CPW1)

    plsc.subcore_barrier()

    db = sid * _DPS
    pltpu.sync_copy(acc.at[pl.ds(db, _DPS)], out_hbm.at[cid, pl.ds(db, _DPS)])


_CPW2 = _EPAD // _CHUNK // _NS


@functools.partial(
    pl.kernel,
    mesh=_mesh,
    compiler_params=_sc_params,
    cost_estimate=_sc_cost,
    out_type=(jax.ShapeDtypeStruct((_ACC, _H), jnp.float32),
              jax.ShapeDtypeStruct((_ACC, _H), jnp.float32)),
    scratch_types=[
        pltpu.VMEM((_CPW2, _CHUNK), jnp.int32),
        pltpu.VMEM((_CPW2, _CHUNK), jnp.int32),
        pltpu.VMEM((_NBUF, _CHUNK, _H), jnp.float32),
        pltpu.VMEM_SHARED((_ACC, _H), jnp.float32),
        pltpu.VMEM_SHARED((1000, _H), jnp.float32),
    ] + [pltpu.SemaphoreType.DMA] * _NBUF,
)
def _sc_edgetime(eenc_hbm, tenc_hbm, sidx_hbm, tidx_hbm, dep_hbm, agge_out,
                 aggt_out, sidx, tidx, rowsb, acc, tencs, *sems):
    cid = lax.axis_index("c")
    sid = lax.axis_index("s")

    rows = [rowsb.at[j] for j in range(_NBUF)]

    @pl.when(jnp.logical_and(cid == 0, sid < _NS - 1))
    def _stage():
        pltpu.sync_copy(tenc_hbm.at[pl.ds(sid * 64, 64)],
                        tencs.at[pl.ds(sid * 64, 64)])

    @pl.when(jnp.logical_and(cid == 0, sid == _NS - 1))
    def _stage_last():
        pltpu.sync_copy(tenc_hbm.at[pl.ds(960, 40)], tencs.at[pl.ds(960, 40)])

    def zfill(i, carry):
        for j in range(_H // 16):
            rowsb[0, i, pl.ds(j * 16, 16)] = _z16()
        return carry
    lax.fori_loop(0, _CHUNK, zfill, 0)

    zb = sid * (_ACC // _NS)

    def zcp(i, carry):
        pltpu.sync_copy(rows[0], acc.at[pl.ds(zb + i * _CHUNK, _CHUNK)])
        return carry
    lax.fori_loop(0, _ZPC, zcp, 0)
    plsc.subcore_barrier()

    pltpu.sync_copy(sidx_hbm.at[pl.ds(sid * _CPW2, _CPW2)], sidx)

    @pl.when(cid == 1)
    def _edge_job():
        emax = eenc_hbm.shape[0] - _CHUNK
        ebase = sid * (_CPW2 * _CHUNK)

        def rd(c):
            return jnp.minimum(ebase + c * _CHUNK, emax)
        for j in range(_NBUF):
            pltpu.async_copy(eenc_hbm.at[pl.ds(rd(j), _CHUNK)], rows[j],
                             sems[j])

        def body(t, carry):
            for j in range(_NBUF):
                c = _NBUF * t + j
                pltpu.make_async_copy(eenc_hbm.at[pl.ds(rd(c), _CHUNK)],
                                      rows[j], sems[j]).wait()
                pltpu.sync_copy(rows[j], acc.at[sidx.at[c]], add=True)

                @pl.when(c + _NBUF < _CPW2)
                def _():
                    pltpu.async_copy(
                        eenc_hbm.at[pl.ds(rd(c + _NBUF), _CHUNK)], rows[j],
                        sems[j])
            return carry
        lax.fori_loop(0, _CPW2 // _NBUF, body, 0)

    @pl.when(cid == 0)
    def _time_job():
        pltpu.sync_copy(tidx_hbm.at[pl.ds(sid * _CPW2, _CPW2)], tidx)
        _pipe_gather_scatter(tencs, tidx, sidx, rows, sems, acc, _CPW2)

    plsc.subcore_barrier()

    db = sid * _DPS

    @pl.when(cid == 1)
    def _dump_e():
        pltpu.sync_copy(acc.at[pl.ds(db, _DPS)], agge_out.at[pl.ds(db, _DPS)])

    @pl.when(cid == 0)
    def _dump_t():
        pltpu.sync_copy(acc.at[pl.ds(db, _DPS)], aggt_out.at[pl.ds(db, _DPS)])


def _enc_body(x_ref, wn_ref, bn_ref, wg1_ref, wres_ref, xe_ref, xw1_ref,
              res_ref):
    xe = jnp.maximum(jnp.dot(x_ref[...], wn_ref[...],
                             preferred_element_type=jnp.float32) + bn_ref[...],
                     0.0)
    xe_ref[...] = xe
    xw1_ref[...] = jnp.dot(xe, wg1_ref[...], preferred_element_type=jnp.float32)
    res_ref[...] = jnp.dot(xe, wres_ref[...], preferred_element_type=jnp.float32)


def _k_enc(x, W_node, b_node, W_g1, W_res):
    n, d = x.shape
    grid = (n // _BN,)
    return pl.pallas_call(
        _enc_body,
        grid=grid,
        in_specs=[
            pl.BlockSpec((_BN, d), lambda i: (i, 0)),
            pl.BlockSpec((d, _H), lambda i: (0, 0)),
            pl.BlockSpec((1, _H), lambda i: (0, 0)),
            pl.BlockSpec((_H, _H), lambda i: (0, 0)),
            pl.BlockSpec((_H, _H), lambda i: (0, 0)),
        ],
        out_specs=[pl.BlockSpec((_BN, _H), lambda i: (i, 0))] * 3,
        out_shape=[jax.ShapeDtypeStruct((n, _H), jnp.float32)] * 3,
    )(x, W_node, b_node, W_g1, W_res)


def _edge_mm_body(aT_ref, w_ref, b_ref, o_ref):
    o_ref[...] = jnp.maximum(
        lax.dot_general(aT_ref[...], w_ref[...], (((0,), (0,)), ((), ())),
                        preferred_element_type=jnp.float32) + b_ref[...], 0.0)


def _k_edge_mm(aT, W, b, blk):
    d, n = aT.shape
    grid = (n // blk,)
    return pl.pallas_call(
        _edge_mm_body,
        grid=grid,
        in_specs=[
            pl.BlockSpec((d, blk), lambda i: (0, i)),
            pl.BlockSpec((d, _H), lambda i: (0, 0)),
            pl.BlockSpec((1, _H), lambda i: (0, 0)),
        ],
        out_specs=pl.BlockSpec((blk, _H), lambda i: (i, 0)),
        out_shape=jax.ShapeDtypeStruct((n, _H), jnp.float32),
    )(aT, W, b)


def _mm_relu_body(a_ref, w_ref, b_ref, o_ref):
    o_ref[...] = jnp.maximum(
        jnp.dot(a_ref[...], w_ref[...], preferred_element_type=jnp.float32)
        + b_ref[...], 0.0)


def _k_mm_relu(a, W, b, blk):
    n, d = a.shape
    grid = (n // blk,)
    return pl.pallas_call(
        _mm_relu_body,
        grid=grid,
        in_specs=[
            pl.BlockSpec((blk, d), lambda i: (i, 0)),
            pl.BlockSpec((d, _H), lambda i: (0, 0)),
            pl.BlockSpec((1, _H), lambda i: (0, 0)),
        ],
        out_specs=pl.BlockSpec((blk, _H), lambda i: (i, 0)),
        out_shape=jax.ShapeDtypeStruct((n, _H), jnp.float32),
    )(a, W, b)


def _prep_body(degp_ref, xw1_ref, xws1_ref):
    deg = degp_ref[0][:, :1] + degp_ref[1][:, :1] + 1.0
    xws1_ref[...] = xw1_ref[...] * lax.rsqrt(deg)


def _k_prep(degp, xw1):
    grid = (_N // _BN,)
    return pl.pallas_call(
        _prep_body,
        grid=grid,
        in_specs=[
            pl.BlockSpec((_NC, _BN, _CW), lambda i: (0, i, 0)),
            pl.BlockSpec((_BN, _H), lambda i: (i, 0)),
        ],
        out_specs=pl.BlockSpec((_BN, _H), lambda i: (i, 0)),
        out_shape=jax.ShapeDtypeStruct((_N, _H), jnp.float32),
    )(degp, xw1)


def _mid1_body(p_ref, degp_ref, xw1_ref, wg2_ref, bg1_ref, xw2_ref, xws2_ref):
    deg = degp_ref[0][:, :1] + degp_ref[1][:, :1] + 1.0
    dis = lax.rsqrt(deg)
    x1 = jnp.maximum((p_ref[0] + p_ref[1]) * dis + xw1_ref[...] / deg
                     + bg1_ref[...], 0.0)
    xw2 = jnp.dot(x1, wg2_ref[...], preferred_element_type=jnp.float32)
    xw2_ref[...] = xw2
    xws2_ref[...] = xw2 * dis


def _k_mid1(out1p, degp, xw1, W_g2, b_g1):
    grid = (_N // _BN,)
    return pl.pallas_call(
        _mid1_body,
        grid=grid,
        in_specs=[
            pl.BlockSpec((_NC, _BN, _H), lambda i: (0, i, 0)),
            pl.BlockSpec((_NC, _BN, _CW), lambda i: (0, i, 0)),
            pl.BlockSpec((_BN, _H), lambda i: (i, 0)),
            pl.BlockSpec((_H, _H), lambda i: (0, 0)),
            pl.BlockSpec((1, _H), lambda i: (0, 0)),
        ],
        out_specs=[pl.BlockSpec((_BN, _H), lambda i: (i, 0))] * 2,
        out_shape=[jax.ShapeDtypeStruct((_N, _H), jnp.float32)] * 2,
    )(out1p, degp, xw1, W_g2, b_g1)


def _mid2_body(p_ref, degp_ref, xw2_ref, res_ref, bg2_ref, bres_ref, x2_ref):
    deg = degp_ref[0][:, :1] + degp_ref[1][:, :1] + 1.0
    dis = lax.rsqrt(deg)
    x2_ref[...] = jnp.maximum(
        (p_ref[0] + p_ref[1]) * dis + xw2_ref[...] / deg + bg2_ref[...]
        + res_ref[...] + bres_ref[...], 0.0)


def _k_mid2(out2p, degp, xw2, res, b_g2, b_res):
    grid = (_N // _BN,)
    return pl.pallas_call(
        _mid2_body,
        grid=grid,
        in_specs=[
            pl.BlockSpec((_NC, _BN, _H), lambda i: (0, i, 0)),
            pl.BlockSpec((_NC, _BN, _CW), lambda i: (0, i, 0)),
            pl.BlockSpec((_BN, _H), lambda i: (i, 0)),
            pl.BlockSpec((_BN, _H), lambda i: (i, 0)),
            pl.BlockSpec((1, _H), lambda i: (0, 0)),
            pl.BlockSpec((1, _H), lambda i: (0, 0)),
        ],
        out_specs=pl.BlockSpec((_BN, _H), lambda i: (i, 0)),
        out_shape=jax.ShapeDtypeStruct((_N, _H), jnp.float32),
    )(out2p, degp, xw2, res, b_g2, b_res)


def _final_body(aggdp_ref, aggep_ref, aggtp_ref, cntp_ref, x2_ref, xe_ref,
                wc1_ref, bc1_ref, wc2_ref, bc2_ref, o_ref):
    scnt = cntp_ref[0][:, :1] + cntp_ref[1][:, :1]
    c = jnp.maximum(scnt, 1.0)
    inv = 1.0 / c
    nf0 = scnt * x2_ref[...] * inv
    nf1 = (aggdp_ref[0] + aggdp_ref[1]) * inv
    nf2 = aggep_ref[...] * inv
    nf3 = aggtp_ref[...] * inv
    s = jnp.sum(jnp.abs(nf0) + jnp.abs(nf1) + jnp.abs(nf2) + jnp.abs(nf3),
                axis=1, keepdims=True)
    m = s < 1e-6
    xe = xe_ref[...]
    nf0 = jnp.where(m, xe, nf0)
    nf1 = jnp.where(m, xe, nf1)
    nf2 = jnp.where(m, xe, nf2)
    nf3 = jnp.where(m, xe, nf3)
    h = (jnp.dot(nf0, wc1_ref[0], preferred_element_type=jnp.float32)
         + jnp.dot(nf1, wc1_ref[1], preferred_element_type=jnp.float32)
         + jnp.dot(nf2, wc1_ref[2], preferred_element_type=jnp.float32)
         + jnp.dot(nf3, wc1_ref[3], preferred_element_type=jnp.float32)
         + bc1_ref[...])
    h = jnp.maximum(h, 0.0)
    o_ref[...] = jnp.dot(h, wc2_ref[...],
                         preferred_element_type=jnp.float32) + bc2_ref[...]


def _k_final(aggdp, aggep, aggtp, cntp, x2, xe, Wc1r, b_c1, Wc2p, bc2p):
    grid = (_N // _BN,)
    return pl.pallas_call(
        _final_body,
        grid=grid,
        in_specs=[
            pl.BlockSpec((_NC, _BN, _H), lambda i: (0, i, 0)),
            pl.BlockSpec((_BN, _H), lambda i: (i, 0)),
            pl.BlockSpec((_BN, _H), lambda i: (i, 0)),
            pl.BlockSpec((_NC, _BN, _CW), lambda i: (0, i, 0)),
            pl.BlockSpec((_BN, _H), lambda i: (i, 0)),
            pl.BlockSpec((_BN, _H), lambda i: (i, 0)),
            pl.BlockSpec((4, _H, _H), lambda i: (0, 0, 0)),
            pl.BlockSpec((1, _H), lambda i: (0, 0)),
            pl.BlockSpec((_H, 128), lambda i: (0, 0)),
            pl.BlockSpec((1, 128), lambda i: (0, 0)),
        ],
        out_specs=pl.BlockSpec((_BN, 128), lambda i: (i, 0)),
        out_shape=jax.ShapeDtypeStruct((_N, 128), jnp.float32),
    )(aggdp, aggep, aggtp, cntp, x2, xe, Wc1r, b_c1, Wc2p, bc2p)


def kernel(x, edge_index, edge_attr, timestamps, W_node, b_node, W_edge,
           b_edge, time_table, W_time, b_time, W_g1, b_g1, W_g2, b_g2, W_res,
           b_res, W_c1, b_c1, W_c2, b_c2):
    E = edge_index.shape[1]
    pad = _EPAD - E
    src = edge_index[0]
    dst = edge_index[1]
    zpad = jnp.zeros((pad,), jnp.int32)
    spad = jnp.full((pad,), _SINK, jnp.int32)
    srcZ = jnp.concatenate([src, zpad]).reshape(-1, _CHUNK)
    srcS = jnp.concatenate([src, spad]).reshape(-1, _CHUNK)
    dstZ = jnp.concatenate([dst, zpad]).reshape(-1, _CHUNK)
    dstS = jnp.concatenate([dst, spad]).reshape(-1, _CHUNK)
    ts = jnp.clip(timestamps, 0, time_table.shape[0] - 1).astype(jnp.int32)
    tsZ = jnp.concatenate([ts, zpad]).reshape(-1, _CHUNK)
    def b2(v):
        return v.reshape(1, -1)

    xe, xw1, res = _k_enc(x, W_node, b2(b_node), W_g1, W_res)
    tenc = _k_mm_relu(time_table, W_time, b2(b_time), time_table.shape[0])
    eenc = _k_edge_mm(edge_attr.T, W_edge, b2(b_edge), 12800)

    degp, cntp = _sc_counts(dstS, srcS)

    xws1 = _k_prep(degp, xw1)
    out1p = _sc_rowpass(xws1, srcZ, dstS)
    xw2, xws2 = _k_mid1(out1p, degp, xw1, W_g2, b2(b_g1))

    out2p = _sc_rowpass(xws2, srcZ, dstS)
    x2 = _k_mid2(out2p, degp, xw2, res, b2(b_g2), b2(b_res))

    aggdp = _sc_rowpass(x2, dstZ, srcS)
    aggep, aggtp = _sc_edgetime(eenc, tenc, srcS, tsZ, aggdp)

    Wc1r = W_c1.reshape(4, _H, _H)
    nout = W_c2.shape[1]
    Wc2p = jnp.zeros((_H, 128), jnp.float32).at[:, :nout].set(W_c2)
    bc2p = jnp.zeros((1, 128), jnp.float32).at[0, :nout].set(b_c2)
    outp = _k_final(aggdp, aggep, aggtp, cntp, x2, xe, Wc1r, b2(b_c1), Wc2p,
                    bc2p)
    return outp[:, :nout]
